# baseline (speedup 1.0000x reference)
import functools

import jax
import jax.numpy as jnp
from jax import lax
from jax.experimental import pallas as pl
from jax.experimental.pallas import tpu as pltpu
from jax.experimental.pallas import tpu_sc as plsc

BM = 128
GATHER_CH = 32
COMBINE_CT = 16


def _router_body(x_ref, gw_ref, logits_ref, wr_ref, dst_ref, meta_ref,
                 c_scr, p_scr):
    x = x_ref[...]
    gw = gw_ref[...]
    logits = lax.dot_general(x, gw, (((1,), (1,)), ((), ())),
                             preferred_element_type=jnp.float32)
    logits_ref[...] = logits
    t, e = logits.shape
    cols = lax.broadcasted_iota(jnp.int32, (t, e), 1)
    m0 = jnp.max(logits, axis=1, keepdims=True)
    e0 = jnp.min(jnp.where(logits == m0, cols, e), axis=1, keepdims=True)
    masked = jnp.where(cols == e0, -jnp.inf, logits)
    m1 = jnp.max(masked, axis=1, keepdims=True)
    e1 = jnp.min(jnp.where(masked == m1, cols, e), axis=1, keepdims=True)
    w0 = jax.nn.sigmoid(m0 - m1)
    wr_ref[...] = jnp.concatenate([w0, 1.0 - w0], axis=1)

    oh0 = (cols == e0).astype(jnp.float32)
    oh1 = (cols == e1).astype(jnp.float32)
    c_scr[...] = oh0 + oh1
    tb = 128
    nb = t // tb
    ri = lax.broadcasted_iota(jnp.int32, (tb, tb), 0)
    ci = lax.broadcasted_iota(jnp.int32, (tb, tb), 1)
    tril = (ci < ri).astype(jnp.float32)

    def blk(i, run):
        cb = c_scr[pl.ds(i * tb, tb), :]
        pb = lax.dot_general(tril, cb, (((1,), (0,)), ((), ())),
                             preferred_element_type=jnp.float32)
        p_scr[pl.ds(i * tb, tb), :] = pb + run
        return run + jnp.sum(cb, axis=0, keepdims=True)

    counts = lax.fori_loop(0, nb, blk, jnp.zeros((1, e), jnp.float32))
    nblk = jnp.floor((counts + (BM - 1)) * (1.0 / BM))
    le = lax.broadcasted_iota(jnp.int32, (e, e), 0)
    ge = lax.broadcasted_iota(jnp.int32, (e, e), 1)
    upper = (le <= ge).astype(jnp.float32)
    ends = lax.dot_general(nblk, upper, (((1,), (0,)), ((), ())),
                           preferred_element_type=jnp.float32)
    pstart = (ends - nblk) * float(BM)
    prefix = p_scr[...]
    base = prefix + pstart
    dst0 = jnp.sum(oh0 * base, axis=1, keepdims=True)
    dst1 = jnp.sum(oh1 * base, axis=1, keepdims=True)
    dst_ref[...] = jnp.concatenate([dst0, dst1], axis=1).astype(jnp.int32)

    meta = jnp.concatenate(
        [nblk, (ends - nblk) * float(BM), jnp.zeros((6, e), jnp.float32)],
        axis=0)
    meta_ref[...] = meta.astype(jnp.int32)


def _router(flat, gate_w):
    t, _ = flat.shape
    e = gate_w.shape[0]
    return pl.pallas_call(
        _router_body,
        out_shape=(
            jax.ShapeDtypeStruct((t, e), jnp.float32),
            jax.ShapeDtypeStruct((t, 2), jnp.float32),
            jax.ShapeDtypeStruct((t, 2), jnp.int32),
            jax.ShapeDtypeStruct((8, e), jnp.int32),
        ),
        scratch_shapes=[
            pltpu.VMEM((t, e), jnp.float32),
            pltpu.VMEM((t, e), jnp.float32),
        ],
    )(flat, gate_w)


def _sc_dispatch(flat, d0, d1, p_pad):
    t, h = flat.shape
    info = plsc.get_sparse_core_info()
    nc = info.num_cores
    nw = nc * info.num_subcores
    tpw = t // nw
    ct = GATHER_CH
    nch = tpw // ct
    assert tpw % ct == 0
    mesh = plsc.VectorSubcoreMesh(core_axis_name="c", subcore_axis_name="s")

    @functools.partial(
        pl.kernel,
        out_type=jax.ShapeDtypeStruct((p_pad, h), jnp.float32),
        mesh=mesh,
        scratch_types=[
            pltpu.VMEM((ct,), jnp.int32),
            pltpu.VMEM((ct,), jnp.int32),
            pltpu.VMEM((ct,), jnp.int32),
            pltpu.VMEM((ct,), jnp.int32),
            pltpu.VMEM((ct, h), jnp.float32),
            pltpu.VMEM((ct, h), jnp.float32),
            pltpu.SemaphoreType.DMA,
            pltpu.SemaphoreType.DMA,
            pltpu.SemaphoreType.DMA,
            pltpu.SemaphoreType.DMA,
            pltpu.SemaphoreType.DMA,
            pltpu.SemaphoreType.DMA,
        ],
    )
    def k(flat_hbm, d0_hbm, d1_hbm, xs_hbm,
          i0a, i0b, i1a, i1b, ra, rb,
          rs0, rs1, s0a, s0b, s1a, s1b):
        wid = lax.axis_index("s") * nc + lax.axis_index("c")
        base = wid * tpw
        ib0 = (i0a, i0b)
        ib1 = (i1a, i1b)
        rbuf = (ra, rb)
        rsem = (rs0, rs1)
        s0 = (s0a, s0b)
        s1 = (s1a, s1b)

        def start(c):
            off = base + c * ct
            r = c % 2
            if c >= 2:
                pltpu.make_async_copy(rbuf[r], xs_hbm.at[ib0[r]], s0[r]).wait()
                pltpu.make_async_copy(rbuf[r], xs_hbm.at[ib1[r]], s1[r]).wait()
            pltpu.async_copy(flat_hbm.at[pl.ds(off, ct)], rbuf[r], rsem[r])
            pltpu.sync_copy(d0_hbm.at[pl.ds(off, ct)], ib0[r])
            pltpu.sync_copy(d1_hbm.at[pl.ds(off, ct)], ib1[r])

        def drain(c):
            off = base + c * ct
            r = c % 2
            pltpu.make_async_copy(
                flat_hbm.at[pl.ds(off, ct)], rbuf[r], rsem[r]).wait()
            pltpu.async_copy(rbuf[r], xs_hbm.at[ib0[r]], s0[r])
            pltpu.async_copy(rbuf[r], xs_hbm.at[ib1[r]], s1[r])

        start(0)
        for c in range(1, nch):
            start(c)
            drain(c - 1)
        drain(nch - 1)
        for c in (max(nch - 2, 0), nch - 1):
            r = c % 2
            pltpu.make_async_copy(rbuf[r], xs_hbm.at[ib0[r]], s0[r]).wait()
            pltpu.make_async_copy(rbuf[r], xs_hbm.at[ib1[r]], s1[r]).wait()

    return k(flat, d0, d1)


def _ffn(xs, wg, wu, wd, nb_arr, ps_arr):
    p_pad, h = xs.shape
    e_num, i_dim, _ = wg.shape

    def body(nb_ref, ps_ref, xs_hbm, wg_hbm, wu_hbm, wd_hbm, ys_hbm,
             wgb, wub, wdb, wsem):
        def wcopies(ei, slot):
            return (
                pltpu.make_async_copy(wg_hbm.at[ei], wgb.at[slot],
                                      wsem.at[slot, 0]),
                pltpu.make_async_copy(wu_hbm.at[ei], wub.at[slot],
                                      wsem.at[slot, 1]),
                pltpu.make_async_copy(wd_hbm.at[ei], wdb.at[slot],
                                      wsem.at[slot, 2]),
            )

        for pe in (0, 1):
            for cp in wcopies(pe, pe % 3):
                cp.start()
        for ei in range(e_num):
            slot = ei % 3
            if ei + 2 < e_num:
                for cp in wcopies(ei + 2, (ei + 2) % 3):
                    cp.start()
            for cp in wcopies(ei, slot):
                cp.wait()
            nb = nb_ref[ei]
            psb = ps_ref[ei] // BM

            def inner(x_ref, y_ref, slot=slot):
                x = x_ref[...]
                a = lax.dot_general(x, wgb[slot], (((1,), (1,)), ((), ())),
                                    preferred_element_type=jnp.float32)
                u = lax.dot_general(x, wub[slot], (((1,), (1,)), ((), ())),
                                    preferred_element_type=jnp.float32)
                hh = a * jax.nn.sigmoid(a) * u
                y_ref[...] = lax.dot_general(
                    hh, wdb[slot], (((1,), (1,)), ((), ())),
                    preferred_element_type=jnp.float32)

            @pl.when(nb > 0)
            def _(nb=nb, psb=psb, inner=inner):
                pltpu.emit_pipeline(
                    inner,
                    grid=(nb,),
                    in_specs=[pl.BlockSpec((BM, h), lambda b: (b + psb, 0))],
                    out_specs=[pl.BlockSpec((BM, h), lambda b: (b + psb, 0))],
                )(xs_hbm, ys_hbm)

    return pl.pallas_call(
        body,
        in_specs=[
            pl.BlockSpec(memory_space=pltpu.SMEM),
            pl.BlockSpec(memory_space=pltpu.SMEM),
            pl.BlockSpec(memory_space=pl.ANY),
            pl.BlockSpec(memory_space=pl.ANY),
            pl.BlockSpec(memory_space=pl.ANY),
            pl.BlockSpec(memory_space=pl.ANY),
        ],
        out_specs=pl.BlockSpec(memory_space=pl.ANY),
        out_shape=jax.ShapeDtypeStruct((p_pad, h), jnp.float32),
        scratch_shapes=[
            pltpu.VMEM((3, i_dim, h), jnp.float32),
            pltpu.VMEM((3, i_dim, h), jnp.float32),
            pltpu.VMEM((3, h, i_dim), jnp.float32),
            pltpu.SemaphoreType.DMA((3, 3)),
        ],
    )(nb_arr, ps_arr, xs, wg, wu, wd)


def _sc_combine(ys, pos0, pos1, w0, w1):
    t = pos0.shape[0]
    h = ys.shape[1]
    info = plsc.get_sparse_core_info()
    nc = info.num_cores
    nw = nc * info.num_subcores
    tpw = t // nw
    ct = COMBINE_CT
    ncc = tpw // ct
    nvec = h // 16
    mesh = plsc.VectorSubcoreMesh(core_axis_name="c", subcore_axis_name="s")

    @functools.partial(
        pl.kernel,
        out_type=jax.ShapeDtypeStruct((t, h), jnp.float32),
        mesh=mesh,
        scratch_types=[
            pltpu.VMEM((tpw + 16,), jnp.float32),
            pltpu.VMEM((tpw + 16,), jnp.float32),
            pltpu.VMEM((ct,), jnp.int32),
            pltpu.VMEM((ct,), jnp.int32),
            pltpu.VMEM((ct,), jnp.int32),
            pltpu.VMEM((ct,), jnp.int32),
            pltpu.VMEM((ct, h), jnp.float32),
            pltpu.VMEM((ct, h), jnp.float32),
            pltpu.VMEM((ct, h), jnp.float32),
            pltpu.VMEM((ct, h), jnp.float32),
            pltpu.SemaphoreType.DMA,
            pltpu.SemaphoreType.DMA,
            pltpu.SemaphoreType.DMA,
            pltpu.SemaphoreType.DMA,
            pltpu.SemaphoreType.DMA,
            pltpu.SemaphoreType.DMA,
        ],
    )
    def k(ys_hbm, p0_hbm, p1_hbm, w0_hbm, w1_hbm, out_hbm,
          w0_v, w1_v, i0a, i0b, i1a, i1b, r0a, r0b, r1a, r1b,
          g0a, g0b, g1a, g1b, sa, sb):
        wid = lax.axis_index("s") * nc + lax.axis_index("c")
        base = wid * tpw
        pltpu.sync_copy(w0_hbm.at[pl.ds(base, tpw)], w0_v.at[pl.ds(0, tpw)])
        pltpu.sync_copy(w1_hbm.at[pl.ds(base, tpw)], w1_v.at[pl.ds(0, tpw)])
        i0 = (i0a, i0b)
        i1 = (i1a, i1b)
        r0 = (r0a, r0b)
        r1 = (r1a, r1b)
        g0 = (g0a, g0b)
        g1 = (g1a, g1b)
        so = (sa, sb)

        def start(c):
            r = c % 2
            off = base + c * ct
            if c >= 2:
                poff = base + (c - 2) * ct
                pltpu.make_async_copy(
                    r0[r], out_hbm.at[pl.ds(poff, ct)], so[r]).wait()
            pltpu.sync_copy(p0_hbm.at[pl.ds(off, ct)], i0[r])
            pltpu.sync_copy(p1_hbm.at[pl.ds(off, ct)], i1[r])
            pltpu.async_copy(ys_hbm.at[i0[r]], r0[r], g0[r])
            pltpu.async_copy(ys_hbm.at[i1[r]], r1[r], g1[r])

        def drain(c):
            r = c % 2
            off = base + c * ct
            pltpu.make_async_copy(ys_hbm.at[i0[r]], r0[r], g0[r]).wait()
            pltpu.make_async_copy(ys_hbm.at[i1[r]], r1[r], g1[r]).wait()

            def tok_body(tt, carry):
                wa = w0_v[pl.ds(c * ct + tt, 16)][0]
                wb = w1_v[pl.ds(c * ct + tt, 16)][0]

                def vec_body(j, carry2):
                    jjb = j * 64
                    for u in range(4):
                        jj = jjb + u * 16
                        r0[r][tt, pl.ds(jj, 16)] = (
                            r0[r][tt, pl.ds(jj, 16)] * wa
                            + r1[r][tt, pl.ds(jj, 16)] * wb
                        )
                    return carry2

                lax.fori_loop(0, nvec // 4, vec_body, 0)
                return carry

            lax.fori_loop(0, ct, tok_body, 0)
            pltpu.async_copy(r0[r], out_hbm.at[pl.ds(off, ct)], so[r])

        start(0)
        for c in range(1, ncc):
            start(c)
            drain(c - 1)
        drain(ncc - 1)
        for c in (ncc - 2, ncc - 1):
            r = c % 2
            off = base + c * ct
            pltpu.make_async_copy(
                r0[r], out_hbm.at[pl.ds(off, ct)], so[r]).wait()

    return k(ys, pos0, pos1, w0, w1)


def kernel(hidden_states, gate_w, Wg, Wu, Wd):
    bsz, seqlen, h = hidden_states.shape
    e = gate_w.shape[0]
    t = bsz * seqlen
    flat = hidden_states.reshape(t, h)

    p_pad = 2 * t + e * BM
    logits, wr, pos2, meta = _router(flat, gate_w)

    xs = _sc_dispatch(flat, pos2[:, 0], pos2[:, 1], p_pad)
    ys = _ffn(xs, Wg, Wu, Wd, meta[0], meta[1])
    final = _sc_combine(ys, pos2[:, 0], pos2[:, 1], wr[:, 0], wr[:, 1])
    return final.reshape(bsz, seqlen, h), logits


# speedup vs baseline: 1.3763x; 1.3763x over previous
import functools

import jax
import jax.numpy as jnp
from jax import lax
from jax.experimental import pallas as pl
from jax.experimental.pallas import tpu as pltpu
from jax.experimental.pallas import tpu_sc as plsc

BM = 256
GATHER_CH = 32
COMBINE_CT = 16


def _router_body(x_ref, gw_ref, logits_ref, wr_ref, dst_ref, be_ref, bv_ref,
                 c_scr, p_scr):
    x = x_ref[...]
    gw = gw_ref[...]
    logits = lax.dot_general(x, gw, (((1,), (1,)), ((), ())),
                             preferred_element_type=jnp.float32)
    logits_ref[...] = logits
    t, e = logits.shape
    cols = lax.broadcasted_iota(jnp.int32, (t, e), 1)
    m0 = jnp.max(logits, axis=1, keepdims=True)
    e0 = jnp.min(jnp.where(logits == m0, cols, e), axis=1, keepdims=True)
    masked = jnp.where(cols == e0, -jnp.inf, logits)
    m1 = jnp.max(masked, axis=1, keepdims=True)
    e1 = jnp.min(jnp.where(masked == m1, cols, e), axis=1, keepdims=True)
    w0 = jax.nn.sigmoid(m0 - m1)
    wr_ref[...] = jnp.concatenate([w0, 1.0 - w0], axis=1)

    oh0 = (cols == e0).astype(jnp.float32)
    oh1 = (cols == e1).astype(jnp.float32)
    c_scr[...] = oh0 + oh1
    tb = 128
    nb = t // tb
    ri = lax.broadcasted_iota(jnp.int32, (tb, tb), 0)
    ci = lax.broadcasted_iota(jnp.int32, (tb, tb), 1)
    tril = (ci < ri).astype(jnp.float32)

    def blk(i, run):
        cb = c_scr[pl.ds(i * tb, tb), :]
        pb = lax.dot_general(tril, cb, (((1,), (0,)), ((), ())),
                             preferred_element_type=jnp.float32)
        p_scr[pl.ds(i * tb, tb), :] = pb + run
        return run + jnp.sum(cb, axis=0, keepdims=True)

    counts = lax.fori_loop(0, nb, blk, jnp.zeros((1, e), jnp.float32))
    nblk = jnp.floor((counts + (BM - 1)) * (1.0 / BM))
    le = lax.broadcasted_iota(jnp.int32, (e, e), 0)
    ge = lax.broadcasted_iota(jnp.int32, (e, e), 1)
    upper = (le <= ge).astype(jnp.float32)
    ends = lax.dot_general(nblk, upper, (((1,), (0,)), ((), ())),
                           preferred_element_type=jnp.float32)
    pstart = (ends - nblk) * float(BM)
    prefix = p_scr[...]
    base = prefix + pstart
    dst0 = jnp.sum(oh0 * base, axis=1, keepdims=True)
    dst1 = jnp.sum(oh1 * base, axis=1, keepdims=True)
    dst_ref[...] = jnp.concatenate([dst0, dst1], axis=1).astype(jnp.int32)

    gmax = be_ref.shape[0]
    grows = lax.broadcasted_iota(jnp.int32, (gmax, e), 0).astype(jnp.float32)
    endsb = jnp.broadcast_to(ends, (gmax, e))
    be = jnp.sum((grows >= endsb).astype(jnp.float32), axis=1, keepdims=True)
    be_ref[...] = jnp.minimum(be, float(e - 1)).astype(jnp.int32)
    total = jnp.max(ends, axis=1, keepdims=True)
    totb = jnp.broadcast_to(total, (gmax, 1))
    gcol = lax.broadcasted_iota(jnp.int32, (gmax, 1), 0).astype(jnp.float32)
    bv_ref[...] = (gcol < totb).astype(jnp.int32)


def _router(flat, gate_w, num_blocks):
    t, _ = flat.shape
    e = gate_w.shape[0]
    return pl.pallas_call(
        _router_body,
        out_shape=(
            jax.ShapeDtypeStruct((t, e), jnp.float32),
            jax.ShapeDtypeStruct((t, 2), jnp.float32),
            jax.ShapeDtypeStruct((t, 2), jnp.int32),
            jax.ShapeDtypeStruct((num_blocks, 1), jnp.int32),
            jax.ShapeDtypeStruct((num_blocks, 1), jnp.int32),
        ),
        scratch_shapes=[
            pltpu.VMEM((t, e), jnp.float32),
            pltpu.VMEM((t, e), jnp.float32),
        ],
    )(flat, gate_w)


def _sc_dispatch(flat, d0, d1, p_pad):
    t, h = flat.shape
    info = plsc.get_sparse_core_info()
    nc = info.num_cores
    nw = nc * info.num_subcores
    tpw = t // nw
    ct = GATHER_CH
    nch = tpw // ct
    assert tpw % ct == 0
    mesh = plsc.VectorSubcoreMesh(core_axis_name="c", subcore_axis_name="s")

    @functools.partial(
        pl.kernel,
        out_type=jax.ShapeDtypeStruct((p_pad, h), jnp.float32),
        mesh=mesh,
        scratch_types=[
            pltpu.VMEM((ct,), jnp.int32),
            pltpu.VMEM((ct,), jnp.int32),
            pltpu.VMEM((ct,), jnp.int32),
            pltpu.VMEM((ct,), jnp.int32),
            pltpu.VMEM((ct, h), jnp.float32),
            pltpu.VMEM((ct, h), jnp.float32),
            pltpu.SemaphoreType.DMA,
            pltpu.SemaphoreType.DMA,
            pltpu.SemaphoreType.DMA,
            pltpu.SemaphoreType.DMA,
            pltpu.SemaphoreType.DMA,
            pltpu.SemaphoreType.DMA,
        ],
    )
    def k(flat_hbm, d0_hbm, d1_hbm, xs_hbm,
          i0a, i0b, i1a, i1b, ra, rb,
          rs0, rs1, s0a, s0b, s1a, s1b):
        wid = lax.axis_index("s") * nc + lax.axis_index("c")
        base = wid * tpw
        ib0 = (i0a, i0b)
        ib1 = (i1a, i1b)
        rbuf = (ra, rb)
        rsem = (rs0, rs1)
        s0 = (s0a, s0b)
        s1 = (s1a, s1b)

        def start(c):
            off = base + c * ct
            r = c % 2
            if c >= 2:
                pltpu.make_async_copy(rbuf[r], xs_hbm.at[ib0[r]], s0[r]).wait()
                pltpu.make_async_copy(rbuf[r], xs_hbm.at[ib1[r]], s1[r]).wait()
            pltpu.async_copy(flat_hbm.at[pl.ds(off, ct)], rbuf[r], rsem[r])
            pltpu.sync_copy(d0_hbm.at[pl.ds(off, ct)], ib0[r])
            pltpu.sync_copy(d1_hbm.at[pl.ds(off, ct)], ib1[r])

        def drain(c):
            off = base + c * ct
            r = c % 2
            pltpu.make_async_copy(
                flat_hbm.at[pl.ds(off, ct)], rbuf[r], rsem[r]).wait()
            pltpu.async_copy(rbuf[r], xs_hbm.at[ib0[r]], s0[r])
            pltpu.async_copy(rbuf[r], xs_hbm.at[ib1[r]], s1[r])

        start(0)
        for c in range(1, nch):
            start(c)
            drain(c - 1)
        drain(nch - 1)
        for c in (max(nch - 2, 0), nch - 1):
            r = c % 2
            pltpu.make_async_copy(rbuf[r], xs_hbm.at[ib0[r]], s0[r]).wait()
            pltpu.make_async_copy(rbuf[r], xs_hbm.at[ib1[r]], s1[r]).wait()

    return k(flat, d0, d1)


def _ffn_body(be_ref, bv_ref, xs_ref, wg0_ref, wg1_ref, wu0_ref, wu1_ref,
              wd0_ref, wd1_ref, ys_ref):
    g = pl.program_id(0)

    @pl.when(bv_ref[g] == 1)
    def _():
        x = xs_ref[...]
        y = None
        for wg_ref, wu_ref, wd_ref in ((wg0_ref, wu0_ref, wd0_ref),
                                       (wg1_ref, wu1_ref, wd1_ref)):
            a = lax.dot_general(x, wg_ref[0], (((1,), (1,)), ((), ())),
                                preferred_element_type=jnp.float32)
            u = lax.dot_general(x, wu_ref[0], (((1,), (1,)), ((), ())),
                                preferred_element_type=jnp.float32)
            h = a * jax.nn.sigmoid(a) * u
            yh = lax.dot_general(h, wd_ref[0], (((1,), (1,)), ((), ())),
                                 preferred_element_type=jnp.float32)
            y = yh if y is None else y + yh
        ys_ref[...] = y


def _ffn(xs, wg, wu, wd, block_expert, block_valid, num_blocks):
    p_pad, h = xs.shape
    i_dim = wg.shape[1]
    ih = i_dim // 2
    grid_spec = pltpu.PrefetchScalarGridSpec(
        num_scalar_prefetch=2,
        grid=(num_blocks,),
        in_specs=[
            pl.BlockSpec((BM, h), lambda g, be, bv: (g, 0)),
            pl.BlockSpec((1, ih, h), lambda g, be, bv: (be[g], 0, 0)),
            pl.BlockSpec((1, ih, h), lambda g, be, bv: (be[g], 1, 0)),
            pl.BlockSpec((1, ih, h), lambda g, be, bv: (be[g], 0, 0)),
            pl.BlockSpec((1, ih, h), lambda g, be, bv: (be[g], 1, 0)),
            pl.BlockSpec((1, h, ih), lambda g, be, bv: (be[g], 0, 0)),
            pl.BlockSpec((1, h, ih), lambda g, be, bv: (be[g], 0, 1)),
        ],
        out_specs=pl.BlockSpec((BM, h), lambda g, be, bv: (g, 0)),
    )
    return pl.pallas_call(
        _ffn_body,
        grid_spec=grid_spec,
        out_shape=jax.ShapeDtypeStruct((p_pad, h), jnp.float32),
    )(block_expert, block_valid, xs, wg, wg, wu, wu, wd, wd)


def _sc_combine(ys, pos0, pos1, w0, w1):
    t = pos0.shape[0]
    h = ys.shape[1]
    info = plsc.get_sparse_core_info()
    nc = info.num_cores
    nw = nc * info.num_subcores
    tpw = t // nw
    ct = COMBINE_CT
    ncc = tpw // ct
    nvec = h // 16
    mesh = plsc.VectorSubcoreMesh(core_axis_name="c", subcore_axis_name="s")

    @functools.partial(
        pl.kernel,
        out_type=jax.ShapeDtypeStruct((t, h), jnp.float32),
        mesh=mesh,
        scratch_types=[
            pltpu.VMEM((tpw + 16,), jnp.float32),
            pltpu.VMEM((tpw + 16,), jnp.float32),
            pltpu.VMEM((ct,), jnp.int32),
            pltpu.VMEM((ct,), jnp.int32),
            pltpu.VMEM((ct,), jnp.int32),
            pltpu.VMEM((ct,), jnp.int32),
            pltpu.VMEM((ct, h), jnp.float32),
            pltpu.VMEM((ct, h), jnp.float32),
            pltpu.VMEM((ct, h), jnp.float32),
            pltpu.VMEM((ct, h), jnp.float32),
            pltpu.SemaphoreType.DMA,
            pltpu.SemaphoreType.DMA,
            pltpu.SemaphoreType.DMA,
            pltpu.SemaphoreType.DMA,
            pltpu.SemaphoreType.DMA,
            pltpu.SemaphoreType.DMA,
        ],
    )
    def k(ys_hbm, p0_hbm, p1_hbm, w0_hbm, w1_hbm, out_hbm,
          w0_v, w1_v, i0a, i0b, i1a, i1b, r0a, r0b, r1a, r1b,
          g0a, g0b, g1a, g1b, sa, sb):
        wid = lax.axis_index("s") * nc + lax.axis_index("c")
        base = wid * tpw
        pltpu.sync_copy(w0_hbm.at[pl.ds(base, tpw)], w0_v.at[pl.ds(0, tpw)])
        pltpu.sync_copy(w1_hbm.at[pl.ds(base, tpw)], w1_v.at[pl.ds(0, tpw)])
        i0 = (i0a, i0b)
        i1 = (i1a, i1b)
        r0 = (r0a, r0b)
        r1 = (r1a, r1b)
        g0 = (g0a, g0b)
        g1 = (g1a, g1b)
        so = (sa, sb)

        def start(c):
            r = c % 2
            off = base + c * ct
            if c >= 2:
                poff = base + (c - 2) * ct
                pltpu.make_async_copy(
                    r0[r], out_hbm.at[pl.ds(poff, ct)], so[r]).wait()
            pltpu.sync_copy(p0_hbm.at[pl.ds(off, ct)], i0[r])
            pltpu.sync_copy(p1_hbm.at[pl.ds(off, ct)], i1[r])
            pltpu.async_copy(ys_hbm.at[i0[r]], r0[r], g0[r])
            pltpu.async_copy(ys_hbm.at[i1[r]], r1[r], g1[r])

        def drain(c):
            r = c % 2
            off = base + c * ct
            pltpu.make_async_copy(ys_hbm.at[i0[r]], r0[r], g0[r]).wait()
            pltpu.make_async_copy(ys_hbm.at[i1[r]], r1[r], g1[r]).wait()

            def tok_body(tt, carry):
                wa = w0_v[pl.ds(c * ct + tt, 16)][0]
                wb = w1_v[pl.ds(c * ct + tt, 16)][0]

                def vec_body(j, carry2):
                    jjb = j * 64
                    for u in range(4):
                        jj = jjb + u * 16
                        r0[r][tt, pl.ds(jj, 16)] = (
                            r0[r][tt, pl.ds(jj, 16)] * wa
                            + r1[r][tt, pl.ds(jj, 16)] * wb
                        )
                    return carry2

                lax.fori_loop(0, nvec // 4, vec_body, 0)
                return carry

            lax.fori_loop(0, ct, tok_body, 0)
            pltpu.async_copy(r0[r], out_hbm.at[pl.ds(off, ct)], so[r])

        start(0)
        for c in range(1, ncc):
            start(c)
            drain(c - 1)
        drain(ncc - 1)
        for c in (ncc - 2, ncc - 1):
            r = c % 2
            off = base + c * ct
            pltpu.make_async_copy(
                r0[r], out_hbm.at[pl.ds(off, ct)], so[r]).wait()

    return k(ys, pos0, pos1, w0, w1)


def kernel(hidden_states, gate_w, Wg, Wu, Wd):
    bsz, seqlen, h = hidden_states.shape
    e = gate_w.shape[0]
    t = bsz * seqlen
    flat = hidden_states.reshape(t, h)

    p_pad = 2 * t + e * BM
    num_blocks = p_pad // BM
    logits, wr, pos2, be2, bv2 = _router(flat, gate_w, num_blocks)
    block_expert, block_valid = be2[:, 0], bv2[:, 0]

    xs = _sc_dispatch(flat, pos2[:, 0], pos2[:, 1], p_pad)
    ys = _ffn(xs, Wg, Wu, Wd, block_expert, block_valid, num_blocks)
    final = _sc_combine(ys, pos2[:, 0], pos2[:, 1], wr[:, 0], wr[:, 1])
    return final.reshape(bsz, seqlen, h), logits


# speedup vs baseline: 1.5357x; 1.1159x over previous
import functools

import jax
import jax.numpy as jnp
from jax import lax
from jax.experimental import pallas as pl
from jax.experimental.pallas import tpu as pltpu
from jax.experimental.pallas import tpu_sc as plsc

BM = 288
GATHER_CH = 32
COMBINE_CT = 16


def _router_body(x_ref, gw_ref, logits_ref, wr_ref, dst_ref, be_ref, bv_ref,
                 c_scr, p_scr):
    x = x_ref[...]
    gw = gw_ref[...]
    logits = lax.dot_general(x, gw, (((1,), (1,)), ((), ())),
                             preferred_element_type=jnp.float32)
    logits_ref[...] = logits
    t, e = logits.shape
    cols = lax.broadcasted_iota(jnp.int32, (t, e), 1)
    m0 = jnp.max(logits, axis=1, keepdims=True)
    e0 = jnp.min(jnp.where(logits == m0, cols, e), axis=1, keepdims=True)
    masked = jnp.where(cols == e0, -jnp.inf, logits)
    m1 = jnp.max(masked, axis=1, keepdims=True)
    e1 = jnp.min(jnp.where(masked == m1, cols, e), axis=1, keepdims=True)
    w0 = jax.nn.sigmoid(m0 - m1)
    wr_ref[...] = jnp.concatenate([w0, 1.0 - w0], axis=1)

    oh0 = (cols == e0).astype(jnp.float32)
    oh1 = (cols == e1).astype(jnp.float32)
    c_scr[...] = oh0 + oh1
    tb = 128
    nb = t // tb
    ri = lax.broadcasted_iota(jnp.int32, (tb, tb), 0)
    ci = lax.broadcasted_iota(jnp.int32, (tb, tb), 1)
    tril = (ci < ri).astype(jnp.float32)

    def blk(i, run):
        cb = c_scr[pl.ds(i * tb, tb), :]
        pb = lax.dot_general(tril, cb, (((1,), (0,)), ((), ())),
                             preferred_element_type=jnp.float32)
        p_scr[pl.ds(i * tb, tb), :] = pb + run
        return run + jnp.sum(cb, axis=0, keepdims=True)

    counts = lax.fori_loop(0, nb, blk, jnp.zeros((1, e), jnp.float32))
    nblk = jnp.floor((counts + (BM - 1)) * (1.0 / BM))
    le = lax.broadcasted_iota(jnp.int32, (e, e), 0)
    ge = lax.broadcasted_iota(jnp.int32, (e, e), 1)
    upper = (le <= ge).astype(jnp.float32)
    ends = lax.dot_general(nblk, upper, (((1,), (0,)), ((), ())),
                           preferred_element_type=jnp.float32)
    pstart = (ends - nblk) * float(BM)
    prefix = p_scr[...]
    base = prefix + pstart
    dst0 = jnp.sum(oh0 * base, axis=1, keepdims=True)
    dst1 = jnp.sum(oh1 * base, axis=1, keepdims=True)
    dst_ref[...] = jnp.concatenate([dst0, dst1], axis=1).astype(jnp.int32)

    gmax = be_ref.shape[0]
    grows = lax.broadcasted_iota(jnp.int32, (gmax, e), 0).astype(jnp.float32)
    endsb = jnp.broadcast_to(ends, (gmax, e))
    be = jnp.sum((grows >= endsb).astype(jnp.float32), axis=1, keepdims=True)
    be_ref[...] = jnp.minimum(be, float(e - 1)).astype(jnp.int32)
    total = jnp.max(ends, axis=1, keepdims=True)
    totb = jnp.broadcast_to(total, (gmax, 1))
    gcol = lax.broadcasted_iota(jnp.int32, (gmax, 1), 0).astype(jnp.float32)
    bv_ref[...] = (gcol < totb).astype(jnp.int32)


def _router(flat, gate_w, num_blocks):
    t, _ = flat.shape
    e = gate_w.shape[0]
    return pl.pallas_call(
        _router_body,
        out_shape=(
            jax.ShapeDtypeStruct((t, e), jnp.float32),
            jax.ShapeDtypeStruct((t, 2), jnp.float32),
            jax.ShapeDtypeStruct((t, 2), jnp.int32),
            jax.ShapeDtypeStruct((num_blocks, 1), jnp.int32),
            jax.ShapeDtypeStruct((num_blocks, 1), jnp.int32),
        ),
        scratch_shapes=[
            pltpu.VMEM((t, e), jnp.float32),
            pltpu.VMEM((t, e), jnp.float32),
        ],
    )(flat, gate_w)


def _sc_dispatch(flat, d0, d1, p_pad):
    t, h = flat.shape
    info = plsc.get_sparse_core_info()
    nc = info.num_cores
    nw = nc * info.num_subcores
    tpw = t // nw
    ct = GATHER_CH
    nch = tpw // ct
    assert tpw % ct == 0
    mesh = plsc.VectorSubcoreMesh(core_axis_name="c", subcore_axis_name="s")

    @functools.partial(
        pl.kernel,
        out_type=jax.ShapeDtypeStruct((p_pad, h), jnp.float32),
        mesh=mesh,
        scratch_types=[
            pltpu.VMEM((ct,), jnp.int32),
            pltpu.VMEM((ct,), jnp.int32),
            pltpu.VMEM((ct,), jnp.int32),
            pltpu.VMEM((ct,), jnp.int32),
            pltpu.VMEM((ct, h), jnp.float32),
            pltpu.VMEM((ct, h), jnp.float32),
            pltpu.SemaphoreType.DMA,
            pltpu.SemaphoreType.DMA,
            pltpu.SemaphoreType.DMA,
            pltpu.SemaphoreType.DMA,
            pltpu.SemaphoreType.DMA,
            pltpu.SemaphoreType.DMA,
        ],
    )
    def k(flat_hbm, d0_hbm, d1_hbm, xs_hbm,
          i0a, i0b, i1a, i1b, ra, rb,
          rs0, rs1, s0a, s0b, s1a, s1b):
        wid = lax.axis_index("s") * nc + lax.axis_index("c")
        base = wid * tpw
        ib0 = (i0a, i0b)
        ib1 = (i1a, i1b)
        rbuf = (ra, rb)
        rsem = (rs0, rs1)
        s0 = (s0a, s0b)
        s1 = (s1a, s1b)

        def start(c):
            off = base + c * ct
            r = c % 2
            if c >= 2:
                pltpu.make_async_copy(rbuf[r], xs_hbm.at[ib0[r]], s0[r]).wait()
                pltpu.make_async_copy(rbuf[r], xs_hbm.at[ib1[r]], s1[r]).wait()
            pltpu.async_copy(flat_hbm.at[pl.ds(off, ct)], rbuf[r], rsem[r])
            pltpu.sync_copy(d0_hbm.at[pl.ds(off, ct)], ib0[r])
            pltpu.sync_copy(d1_hbm.at[pl.ds(off, ct)], ib1[r])

        def drain(c):
            off = base + c * ct
            r = c % 2
            pltpu.make_async_copy(
                flat_hbm.at[pl.ds(off, ct)], rbuf[r], rsem[r]).wait()
            pltpu.async_copy(rbuf[r], xs_hbm.at[ib0[r]], s0[r])
            pltpu.async_copy(rbuf[r], xs_hbm.at[ib1[r]], s1[r])

        start(0)
        for c in range(1, nch):
            start(c)
            drain(c - 1)
        drain(nch - 1)
        for c in (max(nch - 2, 0), nch - 1):
            r = c % 2
            pltpu.make_async_copy(rbuf[r], xs_hbm.at[ib0[r]], s0[r]).wait()
            pltpu.make_async_copy(rbuf[r], xs_hbm.at[ib1[r]], s1[r]).wait()

    return k(flat, d0, d1)


def _ffn_body(be_ref, bv_ref, xs_ref, wg0_ref, wg1_ref, wu0_ref, wu1_ref,
              wd0_ref, wd1_ref, ys_ref):
    g = pl.program_id(0)

    @pl.when(bv_ref[g] == 1)
    def _():
        x = xs_ref[...]
        y = None
        for wg_ref, wu_ref, wd_ref in ((wg0_ref, wu0_ref, wd0_ref),
                                       (wg1_ref, wu1_ref, wd1_ref)):
            a = lax.dot_general(x, wg_ref[0], (((1,), (1,)), ((), ())),
                                preferred_element_type=jnp.float32)
            u = lax.dot_general(x, wu_ref[0], (((1,), (1,)), ((), ())),
                                preferred_element_type=jnp.float32)
            h = a * jax.nn.sigmoid(a) * u
            yh = lax.dot_general(h, wd_ref[0], (((1,), (1,)), ((), ())),
                                 preferred_element_type=jnp.float32)
            y = yh if y is None else y + yh
        ys_ref[...] = y


def _ffn(xs, wg, wu, wd, block_expert, block_valid, num_blocks):
    p_pad, h = xs.shape
    i_dim = wg.shape[1]
    ih = i_dim // 2
    grid_spec = pltpu.PrefetchScalarGridSpec(
        num_scalar_prefetch=2,
        grid=(num_blocks,),
        in_specs=[
            pl.BlockSpec((BM, h), lambda g, be, bv: (g, 0)),
            pl.BlockSpec((1, ih, h), lambda g, be, bv: (be[g], 0, 0)),
            pl.BlockSpec((1, ih, h), lambda g, be, bv: (be[g], 1, 0)),
            pl.BlockSpec((1, ih, h), lambda g, be, bv: (be[g], 0, 0)),
            pl.BlockSpec((1, ih, h), lambda g, be, bv: (be[g], 1, 0)),
            pl.BlockSpec((1, h, ih), lambda g, be, bv: (be[g], 0, 0)),
            pl.BlockSpec((1, h, ih), lambda g, be, bv: (be[g], 0, 1)),
        ],
        out_specs=pl.BlockSpec((BM, h), lambda g, be, bv: (g, 0)),
    )
    return pl.pallas_call(
        _ffn_body,
        grid_spec=grid_spec,
        out_shape=jax.ShapeDtypeStruct((p_pad, h), jnp.float32),
    )(block_expert, block_valid, xs, wg, wg, wu, wu, wd, wd)


def _sc_combine(ys, pos0, pos1, w0, w1):
    t = pos0.shape[0]
    h = ys.shape[1]
    info = plsc.get_sparse_core_info()
    nc = info.num_cores
    nw = nc * info.num_subcores
    tpw = t // nw
    ct = COMBINE_CT
    ncc = tpw // ct
    nvec = h // 16
    mesh = plsc.VectorSubcoreMesh(core_axis_name="c", subcore_axis_name="s")

    @functools.partial(
        pl.kernel,
        out_type=jax.ShapeDtypeStruct((t, h), jnp.float32),
        mesh=mesh,
        scratch_types=[
            pltpu.VMEM((tpw + 16,), jnp.float32),
            pltpu.VMEM((tpw + 16,), jnp.float32),
            pltpu.VMEM((ct,), jnp.int32),
            pltpu.VMEM((ct,), jnp.int32),
            pltpu.VMEM((ct,), jnp.int32),
            pltpu.VMEM((ct,), jnp.int32),
            pltpu.VMEM((ct, h), jnp.float32),
            pltpu.VMEM((ct, h), jnp.float32),
            pltpu.VMEM((ct, h), jnp.float32),
            pltpu.VMEM((ct, h), jnp.float32),
            pltpu.SemaphoreType.DMA,
            pltpu.SemaphoreType.DMA,
            pltpu.SemaphoreType.DMA,
            pltpu.SemaphoreType.DMA,
            pltpu.SemaphoreType.DMA,
            pltpu.SemaphoreType.DMA,
        ],
    )
    def k(ys_hbm, p0_hbm, p1_hbm, w0_hbm, w1_hbm, out_hbm,
          w0_v, w1_v, i0a, i0b, i1a, i1b, r0a, r0b, r1a, r1b,
          g0a, g0b, g1a, g1b, sa, sb):
        wid = lax.axis_index("s") * nc + lax.axis_index("c")
        base = wid * tpw
        pltpu.sync_copy(w0_hbm.at[pl.ds(base, tpw)], w0_v.at[pl.ds(0, tpw)])
        pltpu.sync_copy(w1_hbm.at[pl.ds(base, tpw)], w1_v.at[pl.ds(0, tpw)])
        i0 = (i0a, i0b)
        i1 = (i1a, i1b)
        r0 = (r0a, r0b)
        r1 = (r1a, r1b)
        g0 = (g0a, g0b)
        g1 = (g1a, g1b)
        so = (sa, sb)

        def start(c):
            r = c % 2
            off = base + c * ct
            if c >= 2:
                poff = base + (c - 2) * ct
                pltpu.make_async_copy(
                    r0[r], out_hbm.at[pl.ds(poff, ct)], so[r]).wait()
            pltpu.sync_copy(p0_hbm.at[pl.ds(off, ct)], i0[r])
            pltpu.sync_copy(p1_hbm.at[pl.ds(off, ct)], i1[r])
            pltpu.async_copy(ys_hbm.at[i0[r]], r0[r], g0[r])
            pltpu.async_copy(ys_hbm.at[i1[r]], r1[r], g1[r])

        def drain(c):
            r = c % 2
            off = base + c * ct
            pltpu.make_async_copy(ys_hbm.at[i0[r]], r0[r], g0[r]).wait()
            pltpu.make_async_copy(ys_hbm.at[i1[r]], r1[r], g1[r]).wait()

            def tok_body(tt, carry):
                wa = w0_v[pl.ds(c * ct + tt, 16)][0]
                wb = w1_v[pl.ds(c * ct + tt, 16)][0]

                def vec_body(j, carry2):
                    jjb = j * 64
                    for u in range(4):
                        jj = jjb + u * 16
                        r0[r][tt, pl.ds(jj, 16)] = (
                            r0[r][tt, pl.ds(jj, 16)] * wa
                            + r1[r][tt, pl.ds(jj, 16)] * wb
                        )
                    return carry2

                lax.fori_loop(0, nvec // 4, vec_body, 0)
                return carry

            lax.fori_loop(0, ct, tok_body, 0)
            pltpu.async_copy(r0[r], out_hbm.at[pl.ds(off, ct)], so[r])

        start(0)
        for c in range(1, ncc):
            start(c)
            drain(c - 1)
        drain(ncc - 1)
        for c in (ncc - 2, ncc - 1):
            r = c % 2
            off = base + c * ct
            pltpu.make_async_copy(
                r0[r], out_hbm.at[pl.ds(off, ct)], so[r]).wait()

    return k(ys, pos0, pos1, w0, w1)


def kernel(hidden_states, gate_w, Wg, Wu, Wd):
    bsz, seqlen, h = hidden_states.shape
    e = gate_w.shape[0]
    t = bsz * seqlen
    flat = hidden_states.reshape(t, h)

    p_pad = 2 * t + e * BM
    num_blocks = p_pad // BM
    logits, wr, pos2, be2, bv2 = _router(flat, gate_w, num_blocks)
    block_expert, block_valid = be2[:, 0], bv2[:, 0]

    xs = _sc_dispatch(flat, pos2[:, 0], pos2[:, 1], p_pad)
    ys = _ffn(xs, Wg, Wu, Wd, block_expert, block_valid, num_blocks)
    final = _sc_combine(ys, pos2[:, 0], pos2[:, 1], wr[:, 0], wr[:, 1])
    return final.reshape(bsz, seqlen, h), logits


# speedup vs baseline: 1.5539x; 1.0118x over previous
import functools

import jax
import jax.numpy as jnp
from jax import lax
from jax.experimental import pallas as pl
from jax.experimental.pallas import tpu as pltpu
from jax.experimental.pallas import tpu_sc as plsc

BM = 288
GATHER_CH = 32
COMBINE_CT = 16


def _router_body(x_ref, gw_ref, logits_ref, wr_ref, dst_ref, be_ref, bv_ref,
                 c_scr, p_scr):
    x = x_ref[...]
    gw = gw_ref[...]
    logits = lax.dot_general(x, gw, (((1,), (1,)), ((), ())),
                             preferred_element_type=jnp.float32)
    logits_ref[...] = logits
    t, e = logits.shape
    cols = lax.broadcasted_iota(jnp.int32, (t, e), 1)
    m0 = jnp.max(logits, axis=1, keepdims=True)
    e0 = jnp.min(jnp.where(logits == m0, cols, e), axis=1, keepdims=True)
    masked = jnp.where(cols == e0, -jnp.inf, logits)
    m1 = jnp.max(masked, axis=1, keepdims=True)
    e1 = jnp.min(jnp.where(masked == m1, cols, e), axis=1, keepdims=True)
    w0 = jax.nn.sigmoid(m0 - m1)
    wr_ref[...] = jnp.concatenate([w0, 1.0 - w0], axis=1)

    oh0 = (cols == e0).astype(jnp.float32)
    oh1 = (cols == e1).astype(jnp.float32)
    c_scr[...] = oh0 + oh1
    tb = 128
    nb = t // tb
    ri = lax.broadcasted_iota(jnp.int32, (tb, tb), 0)
    ci = lax.broadcasted_iota(jnp.int32, (tb, tb), 1)
    tril = (ci < ri).astype(jnp.float32)

    def blk(i, run):
        cb = c_scr[pl.ds(i * tb, tb), :]
        pb = lax.dot_general(tril, cb, (((1,), (0,)), ((), ())),
                             preferred_element_type=jnp.float32)
        p_scr[pl.ds(i * tb, tb), :] = pb + run
        return run + jnp.sum(cb, axis=0, keepdims=True)

    counts = lax.fori_loop(0, nb, blk, jnp.zeros((1, e), jnp.float32))
    nblk = jnp.floor((counts + (BM - 1)) * (1.0 / BM))
    le = lax.broadcasted_iota(jnp.int32, (e, e), 0)
    ge = lax.broadcasted_iota(jnp.int32, (e, e), 1)
    upper = (le <= ge).astype(jnp.float32)
    ends = lax.dot_general(nblk, upper, (((1,), (0,)), ((), ())),
                           preferred_element_type=jnp.float32)
    pstart = (ends - nblk) * float(BM)
    prefix = p_scr[...]
    base = prefix + pstart
    dst0 = jnp.sum(oh0 * base, axis=1, keepdims=True)
    dst1 = jnp.sum(oh1 * base, axis=1, keepdims=True)
    dst_ref[...] = jnp.concatenate([dst0, dst1], axis=1).astype(jnp.int32)

    gmax = be_ref.shape[0]
    grows = lax.broadcasted_iota(jnp.int32, (gmax, e), 0).astype(jnp.float32)
    endsb = jnp.broadcast_to(ends, (gmax, e))
    be = jnp.sum((grows >= endsb).astype(jnp.float32), axis=1, keepdims=True)
    be_ref[...] = jnp.minimum(be, float(e - 1)).astype(jnp.int32)
    total = jnp.max(ends, axis=1, keepdims=True)
    totb = jnp.broadcast_to(total, (gmax, 1))
    gcol = lax.broadcasted_iota(jnp.int32, (gmax, 1), 0).astype(jnp.float32)
    bv_ref[...] = (gcol < totb).astype(jnp.int32)


def _router(flat, gate_w, num_blocks):
    t, _ = flat.shape
    e = gate_w.shape[0]
    return pl.pallas_call(
        _router_body,
        out_shape=(
            jax.ShapeDtypeStruct((t, e), jnp.float32),
            jax.ShapeDtypeStruct((t, 2), jnp.float32),
            jax.ShapeDtypeStruct((t, 2), jnp.int32),
            jax.ShapeDtypeStruct((num_blocks, 1), jnp.int32),
            jax.ShapeDtypeStruct((num_blocks, 1), jnp.int32),
        ),
        scratch_shapes=[
            pltpu.VMEM((t, e), jnp.float32),
            pltpu.VMEM((t, e), jnp.float32),
        ],
    )(flat, gate_w)


def _sc_dispatch(flat, d0, d1, p_pad):
    t, h = flat.shape
    info = plsc.get_sparse_core_info()
    nc = info.num_cores
    nw = nc * info.num_subcores
    tpw = t // nw
    ct = GATHER_CH
    nch = tpw // ct
    assert tpw % ct == 0
    mesh = plsc.VectorSubcoreMesh(core_axis_name="c", subcore_axis_name="s")

    @functools.partial(
        pl.kernel,
        out_type=jax.ShapeDtypeStruct((p_pad, h), jnp.float32),
        mesh=mesh,
        scratch_types=[
            pltpu.VMEM((ct,), jnp.int32),
            pltpu.VMEM((ct,), jnp.int32),
            pltpu.VMEM((ct,), jnp.int32),
            pltpu.VMEM((ct,), jnp.int32),
            pltpu.VMEM((ct, h), jnp.float32),
            pltpu.VMEM((ct, h), jnp.float32),
            pltpu.SemaphoreType.DMA,
            pltpu.SemaphoreType.DMA,
            pltpu.SemaphoreType.DMA,
            pltpu.SemaphoreType.DMA,
            pltpu.SemaphoreType.DMA,
            pltpu.SemaphoreType.DMA,
        ],
    )
    def k(flat_hbm, d0_hbm, d1_hbm, xs_hbm,
          i0a, i0b, i1a, i1b, ra, rb,
          rs0, rs1, s0a, s0b, s1a, s1b):
        wid = lax.axis_index("s") * nc + lax.axis_index("c")
        base = wid * tpw
        ib0 = (i0a, i0b)
        ib1 = (i1a, i1b)
        rbuf = (ra, rb)
        rsem = (rs0, rs1)
        s0 = (s0a, s0b)
        s1 = (s1a, s1b)

        def start(c):
            off = base + c * ct
            r = c % 2
            if c >= 2:
                pltpu.make_async_copy(rbuf[r], xs_hbm.at[ib0[r]], s0[r]).wait()
                pltpu.make_async_copy(rbuf[r], xs_hbm.at[ib1[r]], s1[r]).wait()
            pltpu.async_copy(flat_hbm.at[pl.ds(off, ct)], rbuf[r], rsem[r])
            pltpu.sync_copy(d0_hbm.at[pl.ds(off, ct)], ib0[r])
            pltpu.sync_copy(d1_hbm.at[pl.ds(off, ct)], ib1[r])

        def drain(c):
            off = base + c * ct
            r = c % 2
            pltpu.make_async_copy(
                flat_hbm.at[pl.ds(off, ct)], rbuf[r], rsem[r]).wait()
            pltpu.async_copy(rbuf[r], xs_hbm.at[ib0[r]], s0[r])
            pltpu.async_copy(rbuf[r], xs_hbm.at[ib1[r]], s1[r])

        start(0)
        for c in range(1, nch):
            start(c)
            drain(c - 1)
        drain(nch - 1)
        for c in (max(nch - 2, 0), nch - 1):
            r = c % 2
            pltpu.make_async_copy(rbuf[r], xs_hbm.at[ib0[r]], s0[r]).wait()
            pltpu.make_async_copy(rbuf[r], xs_hbm.at[ib1[r]], s1[r]).wait()

    return k(flat, d0, d1)


NWS = 3


def _ffn_body(be_ref, bv_ref, *refs):
    xs_ref = refs[0]
    wg_refs = refs[1:1 + NWS]
    wu_refs = refs[1 + NWS:1 + 2 * NWS]
    wd_refs = refs[1 + 2 * NWS:1 + 3 * NWS]
    ys_ref = refs[1 + 3 * NWS]
    g = pl.program_id(0)

    @pl.when(bv_ref[g] == 1)
    def _():
        x = xs_ref[...]
        y = None
        for wg_ref, wu_ref, wd_ref in zip(wg_refs, wu_refs, wd_refs):
            a = lax.dot_general(x, wg_ref[0], (((1,), (1,)), ((), ())),
                                preferred_element_type=jnp.float32)
            u = lax.dot_general(x, wu_ref[0], (((1,), (1,)), ((), ())),
                                preferred_element_type=jnp.float32)
            h = a * jax.nn.sigmoid(a) * u
            yh = lax.dot_general(h, wd_ref[0], (((1,), (1,)), ((), ())),
                                 preferred_element_type=jnp.float32)
            y = yh if y is None else y + yh
        ys_ref[...] = y


def _ffn(xs, wg, wu, wd, block_expert, block_valid, num_blocks):
    p_pad, h = xs.shape
    i_dim = wg.shape[1]
    ih = i_dim // NWS
    gu_specs = [pl.BlockSpec((1, ih, h), lambda g, be, bv, q=q: (be[g], q, 0))
                for q in range(NWS)]
    d_specs = [pl.BlockSpec((1, h, ih), lambda g, be, bv, q=q: (be[g], 0, q))
               for q in range(NWS)]
    grid_spec = pltpu.PrefetchScalarGridSpec(
        num_scalar_prefetch=2,
        grid=(num_blocks,),
        in_specs=[pl.BlockSpec((BM, h), lambda g, be, bv: (g, 0))]
        + gu_specs + gu_specs + d_specs,
        out_specs=pl.BlockSpec((BM, h), lambda g, be, bv: (g, 0)),
    )
    return pl.pallas_call(
        _ffn_body,
        grid_spec=grid_spec,
        out_shape=jax.ShapeDtypeStruct((p_pad, h), jnp.float32),
    )(block_expert, block_valid, xs,
      *([wg] * NWS), *([wu] * NWS), *([wd] * NWS))


def _sc_combine(ys, pos0, pos1, w0, w1):
    t = pos0.shape[0]
    h = ys.shape[1]
    info = plsc.get_sparse_core_info()
    nc = info.num_cores
    nw = nc * info.num_subcores
    tpw = t // nw
    ct = COMBINE_CT
    ncc = tpw // ct
    nvec = h // 16
    mesh = plsc.VectorSubcoreMesh(core_axis_name="c", subcore_axis_name="s")

    @functools.partial(
        pl.kernel,
        out_type=jax.ShapeDtypeStruct((t, h), jnp.float32),
        mesh=mesh,
        scratch_types=[
            pltpu.VMEM((tpw + 16,), jnp.float32),
            pltpu.VMEM((tpw + 16,), jnp.float32),
            pltpu.VMEM((ct,), jnp.int32),
            pltpu.VMEM((ct,), jnp.int32),
            pltpu.VMEM((ct,), jnp.int32),
            pltpu.VMEM((ct,), jnp.int32),
            pltpu.VMEM((ct, h), jnp.float32),
            pltpu.VMEM((ct, h), jnp.float32),
            pltpu.VMEM((ct, h), jnp.float32),
            pltpu.VMEM((ct, h), jnp.float32),
            pltpu.SemaphoreType.DMA,
            pltpu.SemaphoreType.DMA,
            pltpu.SemaphoreType.DMA,
            pltpu.SemaphoreType.DMA,
            pltpu.SemaphoreType.DMA,
            pltpu.SemaphoreType.DMA,
        ],
    )
    def k(ys_hbm, p0_hbm, p1_hbm, w0_hbm, w1_hbm, out_hbm,
          w0_v, w1_v, i0a, i0b, i1a, i1b, r0a, r0b, r1a, r1b,
          g0a, g0b, g1a, g1b, sa, sb):
        wid = lax.axis_index("s") * nc + lax.axis_index("c")
        base = wid * tpw
        pltpu.sync_copy(w0_hbm.at[pl.ds(base, tpw)], w0_v.at[pl.ds(0, tpw)])
        pltpu.sync_copy(w1_hbm.at[pl.ds(base, tpw)], w1_v.at[pl.ds(0, tpw)])
        i0 = (i0a, i0b)
        i1 = (i1a, i1b)
        r0 = (r0a, r0b)
        r1 = (r1a, r1b)
        g0 = (g0a, g0b)
        g1 = (g1a, g1b)
        so = (sa, sb)

        def start(c):
            r = c % 2
            off = base + c * ct
            if c >= 2:
                poff = base + (c - 2) * ct
                pltpu.make_async_copy(
                    r0[r], out_hbm.at[pl.ds(poff, ct)], so[r]).wait()
            pltpu.sync_copy(p0_hbm.at[pl.ds(off, ct)], i0[r])
            pltpu.sync_copy(p1_hbm.at[pl.ds(off, ct)], i1[r])
            pltpu.async_copy(ys_hbm.at[i0[r]], r0[r], g0[r])
            pltpu.async_copy(ys_hbm.at[i1[r]], r1[r], g1[r])

        def drain(c):
            r = c % 2
            off = base + c * ct
            pltpu.make_async_copy(ys_hbm.at[i0[r]], r0[r], g0[r]).wait()
            pltpu.make_async_copy(ys_hbm.at[i1[r]], r1[r], g1[r]).wait()

            def tok_body(tt, carry):
                wa = w0_v[pl.ds(c * ct + tt, 16)][0]
                wb = w1_v[pl.ds(c * ct + tt, 16)][0]

                def vec_body(j, carry2):
                    jjb = j * 64
                    for u in range(4):
                        jj = jjb + u * 16
                        r0[r][tt, pl.ds(jj, 16)] = (
                            r0[r][tt, pl.ds(jj, 16)] * wa
                            + r1[r][tt, pl.ds(jj, 16)] * wb
                        )
                    return carry2

                lax.fori_loop(0, nvec // 4, vec_body, 0)
                return carry

            lax.fori_loop(0, ct, tok_body, 0)
            pltpu.async_copy(r0[r], out_hbm.at[pl.ds(off, ct)], so[r])

        start(0)
        for c in range(1, ncc):
            start(c)
            drain(c - 1)
        drain(ncc - 1)
        for c in (ncc - 2, ncc - 1):
            r = c % 2
            off = base + c * ct
            pltpu.make_async_copy(
                r0[r], out_hbm.at[pl.ds(off, ct)], so[r]).wait()

    return k(ys, pos0, pos1, w0, w1)


def kernel(hidden_states, gate_w, Wg, Wu, Wd):
    bsz, seqlen, h = hidden_states.shape
    e = gate_w.shape[0]
    t = bsz * seqlen
    flat = hidden_states.reshape(t, h)

    p_pad = 2 * t + e * BM
    num_blocks = p_pad // BM
    logits, wr, pos2, be2, bv2 = _router(flat, gate_w, num_blocks)
    block_expert, block_valid = be2[:, 0], bv2[:, 0]

    xs = _sc_dispatch(flat, pos2[:, 0], pos2[:, 1], p_pad)
    ys = _ffn(xs, Wg, Wu, Wd, block_expert, block_valid, num_blocks)
    final = _sc_combine(ys, pos2[:, 0], pos2[:, 1], wr[:, 0], wr[:, 1])
    return final.reshape(bsz, seqlen, h), logits
